# baseline (speedup 1.0000x reference)
import functools

import jax
import jax.numpy as jnp
import numpy as np
from jax import lax
from jax.experimental import pallas as pl
from jax.experimental.pallas import tpu as pltpu
from jax.experimental.pallas import tpu_sc as plsc

N_ACT = 16384
N_PAS = 8192
TOT = N_ACT + N_PAS
FEAT = 256
HEADS = 8
MAPS = 4
BATCH = 4
_S = (64, 32, 16, 8)
_BASES = (0, 16384, 20480, 21504)
_IDTOT = 21760

_NC, _NS = 2, 16
_NW = _NC * _NS
_C = 8
_RPT = N_ACT // _NW
_NCH = _RPT // _C

_LANE = np.arange(128)
_MSL = _LANE % 16
_MJ = _MSL // 4
_SJ = _MSL % 4
_SM128 = np.asarray(_S, np.int32)[_MJ]
_BASE128 = np.asarray(_BASES, np.int32)[_MJ]
_HW128 = (_SM128.astype(np.int64) ** 2).astype(np.int32)
_DX128 = (_SJ % 2).astype(np.int32)
_DY128 = (_SJ // 2).astype(np.int32)

_P_A = np.zeros((128, 128), np.float32)
for _j in range(128):
    _P_A[8 * (_j // 32) + (_j % 8), _j] = 1.0
_P_S = np.zeros((128, 128), np.float32)
for _j in range(128):
    _P_S[4 * (_j // 32) + ((_j // 8) % 4), _j] = 1.0
_A_H = np.zeros((128, 128), np.float32)
for _j in range(128):
    for _l in range(32):
        if _l % 8 == _j % 8:
            _A_H[_l, _j] = 1.0

_OUTPERM = np.empty((256,), np.int32)
for _q in range(256):
    _v, _r = _q // 32, _q % 32
    _OUTPERM[_q] = _v * 32 + (2 * _r if _r < 16 else 2 * (_r - 16) + 1)


def _matmul_body(x_ref, w_ref, b_ref, o_ref):
    o_ref[...] = (
        jnp.dot(x_ref[...], w_ref[...], preferred_element_type=jnp.float32)
        + b_ref[...]
    )


def _matmul(x, w_t, b, blk=512):
    n, k = x.shape
    m = w_t.shape[1]
    return pl.pallas_call(
        _matmul_body,
        grid=(n // blk,),
        in_specs=[
            pl.BlockSpec((blk, k), lambda i: (i, 0)),
            pl.BlockSpec((k, m), lambda i: (0, 0)),
            pl.BlockSpec((1, m), lambda i: (0, 0)),
        ],
        out_specs=pl.BlockSpec((blk, m), lambda i: (i, 0)),
        out_shape=jax.ShapeDtypeStruct((n, m), jnp.float32),
    )(x, w_t, b)


_NBLK_ACT = N_ACT // 512
_NBLK_TOT = TOT // 512


def _fused_body(act_ref, pas_ref, ids_ref, se_ref, vwt_ref, vb_ref,
                awt_ref, ab_ref, pa_ref, ps_ref, ah_ref,
                vals_ref, offs_ref, fw_ref):
    pid = pl.program_id(0)

    @pl.when(pid < _NBLK_ACT)
    def _():
        x = act_ref[...]
        vals_ref[...] = (
            jnp.dot(x, vwt_ref[...], preferred_element_type=jnp.float32)
            + vb_ref[...]
        ).astype(jnp.bfloat16)
        ids = ids_ref[...]
        r = x.shape[0]
        b = ids[:, 0:1]
        g = ids[:, 1:2]
        gx = ids[:, 2:3]
        gy = ids[:, 3:4]

        gb = jnp.broadcast_to(g, (r, FEAT))
        se = se_ref[...]
        af = x
        for mm in range(MAPS):
            af = af + jnp.where(
                gb == mm, jnp.broadcast_to(se[mm : mm + 1, :], (r, FEAT)), 0.0
            )

        l = jnp.dot(af, awt_ref[...], preferred_element_type=jnp.float32) + ab_ref[...]
        e = jnp.exp(l - jnp.max(l, axis=1, keepdims=True))
        ssum = jnp.dot(e, ah_ref[...], preferred_element_type=jnp.float32,
                       precision=lax.Precision.HIGHEST)
        aw = e / ssum

        lane = lax.broadcasted_iota(jnp.int32, (r, 128), 1)
        msl = lane % 16
        mj = msl // 4
        sj = msl % 4
        dx = sj % 2
        dy = sj // 2

        def selm(idx, vals):
            out = jnp.full((r, 128), vals[MAPS - 1], jnp.int32)
            for mm in range(MAPS - 1):
                out = jnp.where(idx == mm, vals[mm], out)
            return out

        smi = selm(mj, _S)
        base = selm(mj, _BASES)
        hw = smi * smi
        g128 = jnp.broadcast_to(g, (r, 128))
        sg = selm(g128, _S)
        sgf = sg.astype(jnp.float32)
        smf = smi.astype(jnp.float32)
        gxf = jnp.broadcast_to(gx, (r, 128)).astype(jnp.float32)
        gyf = jnp.broadcast_to(gy, (r, 128)).astype(jnp.float32)
        sx = (gxf + 0.5) / sgf * smf - 0.5
        sy = (gyf + 0.5) / sgf * smf - 0.5
        x0 = jnp.floor(sx).astype(jnp.int32)
        y0 = jnp.floor(sy).astype(jnp.int32)
        xi = x0 + dx
        yi = y0 + dy
        wx = 1.0 - jnp.abs(sx - xi.astype(jnp.float32))
        wy = 1.0 - jnp.abs(sy - yi.astype(jnp.float32))
        sw = wx * wy
        xc = jnp.clip(xi, 0, smi - 1)
        yc = jnp.clip(yi, 0, smi - 1)
        offs = base + jnp.broadcast_to(b, (r, 128)) * hw + yc * smi + xc
        offs_ref[...] = offs[:, :16]

        aw_p = jnp.dot(aw, pa_ref[...], preferred_element_type=jnp.float32,
                       precision=lax.Precision.HIGHEST)
        sw_p = jnp.dot(sw, ps_ref[...], preferred_element_type=jnp.float32,
                       precision=lax.Precision.HIGHEST)
        fw_ref[...] = aw_p * sw_p

    @pl.when(pid >= _NBLK_ACT)
    def _():
        vals_ref[...] = (
            jnp.dot(pas_ref[...], vwt_ref[...], preferred_element_type=jnp.float32)
            + vb_ref[...]
        ).astype(jnp.bfloat16)


def _fused_tc(act, pas, ids, se, vwt, vb, awt, ab):
    return pl.pallas_call(
        _fused_body,
        grid=(_NBLK_TOT,),
        in_specs=[
            pl.BlockSpec((512, FEAT), lambda i: (jnp.minimum(i, _NBLK_ACT - 1), 0)),
            pl.BlockSpec((512, FEAT), lambda i: (jnp.maximum(i - _NBLK_ACT, 0), 0)),
            pl.BlockSpec((512, 4), lambda i: (jnp.minimum(i, _NBLK_ACT - 1), 0)),
            pl.BlockSpec((MAPS, FEAT), lambda i: (0, 0)),
            pl.BlockSpec((FEAT, FEAT), lambda i: (0, 0)),
            pl.BlockSpec((1, FEAT), lambda i: (0, 0)),
            pl.BlockSpec((FEAT, 128), lambda i: (0, 0)),
            pl.BlockSpec((1, 128), lambda i: (0, 0)),
            pl.BlockSpec((128, 128), lambda i: (0, 0)),
            pl.BlockSpec((128, 128), lambda i: (0, 0)),
            pl.BlockSpec((128, 128), lambda i: (0, 0)),
        ],
        out_specs=[
            pl.BlockSpec((512, FEAT), lambda i: (i, 0)),
            pl.BlockSpec((512, 16), lambda i: (jnp.minimum(i, _NBLK_ACT - 1), 0)),
            pl.BlockSpec((512, 128), lambda i: (jnp.minimum(i, _NBLK_ACT - 1), 0)),
        ],
        out_shape=[
            jax.ShapeDtypeStruct((TOT, FEAT), jnp.bfloat16),
            jax.ShapeDtypeStruct((N_ACT, 16), jnp.int32),
            jax.ShapeDtypeStruct((N_ACT, 128), jnp.float32),
        ],
    )(act, pas, ids, se, vwt, vb, awt, ab, _P_A, _P_S, _A_H)


def _sc_compute_chunk(rows_v, fw_v, out_v, out_hbm, fb):
    for f in range(_C):
        def ms_body(msp, acc):
            accl = list(acc)
            fwvec = fw_v[f, pl.ds(msp * 16, 16)]
            for ms_off in range(2):
                rr = f * 16 + msp * 2 + ms_off
                for h in range(HEADS):
                    w = fwvec[ms_off * 8 + h]
                    xi = rows_v[rr, pl.ds(h * 16, 16)]
                    xe = plsc.bitcast(xi << 16, jnp.float32)
                    xo = plsc.bitcast(xi & jnp.int32(-65536), jnp.float32)
                    accl[2 * h] = accl[2 * h] + w * xe
                    accl[2 * h + 1] = accl[2 * h + 1] + w * xo
            return tuple(accl)

        acc = lax.fori_loop(
            0, 8, ms_body,
            tuple(jnp.zeros((16,), jnp.float32) for _ in range(16)),
        )
        for v in range(16):
            out_v[f, pl.ds(v * 16, 16)] = acc[v]
    pltpu.sync_copy(out_v, out_hbm.at[pl.ds(fb, _C)])


def _sc_body(vals_hbm, fw_hbm, offs_hbm, idmap_hbm, out_hbm,
             idmap_v, offs_v, fid_v, fw_va, fw_vb, rows_va, rows_vb, out_v,
             sem_ra, sem_rb, sem_fa, sem_fb):
    wid = lax.axis_index("s") * _NC + lax.axis_index("c")
    base = wid * _RPT
    pltpu.sync_copy(idmap_hbm, idmap_v)
    pltpu.sync_copy(offs_hbm.at[pl.ds(base * 16, _RPT * 16)], offs_v)

    def fid_loop(f, c):
        fid = plsc.load_gather(idmap_v, [offs_v[pl.ds(f * 16, 16)]])
        fid_v[pl.ds(f * 16, 16)] = fid
        return c

    lax.fori_loop(0, _RPT, fid_loop, 0)

    def start(k, rows_v, fw_v, sem_r, sem_f):
        pltpu.async_copy(
            vals_hbm.at[fid_v.at[pl.ds(k * (_C * 16), _C * 16)]], rows_v, sem_r
        )
        pltpu.async_copy(fw_hbm.at[pl.ds(base + k * _C, _C)], fw_v, sem_f)

    def wait(k, rows_v, fw_v, sem_r, sem_f):
        pltpu.make_async_copy(
            vals_hbm.at[fid_v.at[pl.ds(k * (_C * 16), _C * 16)]], rows_v, sem_r
        ).wait()
        pltpu.make_async_copy(
            fw_hbm.at[pl.ds(base + k * _C, _C)], fw_v, sem_f
        ).wait()

    start(0, rows_va, fw_va, sem_ra, sem_fa)

    def pair(g, carry):
        ka = 2 * g
        kb = 2 * g + 1
        start(kb, rows_vb, fw_vb, sem_rb, sem_fb)
        wait(ka, rows_va, fw_va, sem_ra, sem_fa)
        _sc_compute_chunk(rows_va, fw_va, out_v, out_hbm, base + ka * _C)

        @pl.when(g < _NCH // 2 - 1)
        def _():
            start(ka + 2, rows_va, fw_va, sem_ra, sem_fa)

        wait(kb, rows_vb, fw_vb, sem_rb, sem_fb)
        _sc_compute_chunk(rows_vb, fw_vb, out_v, out_hbm, base + kb * _C)
        return carry

    lax.fori_loop(0, _NCH // 2, pair, 0)


def _sc_gather_combine(all_vals, fw, offs, flat_idmap):
    mesh = plsc.VectorSubcoreMesh(
        core_axis_name="c", subcore_axis_name="s", num_cores=_NC, num_subcores=_NS
    )
    run = functools.partial(
        pl.kernel,
        out_type=jax.ShapeDtypeStruct((N_ACT, FEAT), jnp.float32),
        mesh=mesh,
        compiler_params=pltpu.CompilerParams(needs_layout_passes=False),
        scratch_types=[
            pltpu.VMEM((_IDTOT,), jnp.int32),
            pltpu.VMEM((_RPT * 16,), jnp.int32),
            pltpu.VMEM((_RPT * 16,), jnp.int32),
            pltpu.VMEM((_C, 128), jnp.float32),
            pltpu.VMEM((_C, 128), jnp.float32),
            pltpu.VMEM((_C * 16, FEAT // 2), jnp.int32),
            pltpu.VMEM((_C * 16, FEAT // 2), jnp.int32),
            pltpu.VMEM((_C, FEAT), jnp.float32),
            pltpu.SemaphoreType.DMA,
            pltpu.SemaphoreType.DMA,
            pltpu.SemaphoreType.DMA,
            pltpu.SemaphoreType.DMA,
        ],
    )(_sc_body)
    return run(all_vals, fw, offs, flat_idmap)


def kernel(in_act_feats, act_batch_ids, act_map_ids, act_xy_ids, map_shapes,
           pas_feats, id_map0, id_map1, id_map2, id_map3, scale_embed,
           attn_W, attn_b, val_W, val_b, out_W, out_b):
    del map_shapes
    ids = jnp.stack(
        [act_batch_ids, act_map_ids, act_xy_ids[:, 0], act_xy_ids[:, 1]], axis=1
    )
    attn_wr_t = attn_W.reshape(HEADS, MAPS, FEAT).transpose(1, 0, 2).reshape(
        HEADS * MAPS, FEAT).T
    awt128 = jnp.concatenate([attn_wr_t] * 4, axis=1)
    attn_br = attn_b.reshape(HEADS, MAPS).T.reshape(1, HEADS * MAPS)
    ab128 = jnp.concatenate([attn_br] * 4, axis=1)
    flat_idmap = jnp.concatenate(
        [m.reshape(-1) for m in (id_map0, id_map1, id_map2, id_map3)]
    )
    all_vals, offs, fw = _fused_tc(
        in_act_feats, pas_feats, ids, scale_embed,
        val_W.T, val_b.reshape(1, FEAT), awt128, ab128,
    )
    vals_i32 = lax.bitcast_convert_type(
        all_vals.reshape(TOT, FEAT // 2, 2), jnp.int32
    )
    val_feats = _sc_gather_combine(vals_i32, fw, offs.reshape(-1), flat_idmap)
    out_wt_perm = out_W.T[jnp.asarray(_OUTPERM), :]
    return _matmul(val_feats, out_wt_perm, out_b.reshape(1, FEAT))


# speedup vs baseline: 1.5285x; 1.5285x over previous
import functools

import jax
import jax.numpy as jnp
import numpy as np
from jax import lax
from jax.experimental import pallas as pl
from jax.experimental.pallas import tpu as pltpu
from jax.experimental.pallas import tpu_sc as plsc

N_ACT = 16384
N_PAS = 8192
TOT = N_ACT + N_PAS
FEAT = 256
HEADS = 8
MAPS = 4
BATCH = 4
_S = (64, 32, 16, 8)
_BASES = (0, 16384, 20480, 21504)
_IDTOT = 21760

_NC, _NS = 2, 16
_NW = _NC * _NS
_C = 8
_RPT = N_ACT // _NW
_NCH = _RPT // _C

_LANE = np.arange(128)
_MSL = _LANE % 16
_MJ = _MSL // 4
_SJ = _MSL % 4
_SM128 = np.asarray(_S, np.int32)[_MJ]
_BASE128 = np.asarray(_BASES, np.int32)[_MJ]
_HW128 = (_SM128.astype(np.int64) ** 2).astype(np.int32)
_DX128 = (_SJ % 2).astype(np.int32)
_DY128 = (_SJ // 2).astype(np.int32)

_P_A = np.zeros((128, 128), np.float32)
for _j in range(128):
    _P_A[8 * (_j // 32) + (_j % 8), _j] = 1.0
_P_S = np.zeros((128, 128), np.float32)
for _j in range(128):
    _P_S[4 * (_j // 32) + ((_j // 8) % 4), _j] = 1.0
_A_H = np.zeros((128, 128), np.float32)
for _j in range(128):
    for _l in range(32):
        if _l % 8 == _j % 8:
            _A_H[_l, _j] = 1.0

_OUTPERM = np.empty((256,), np.int32)
for _q in range(256):
    _v, _r = _q // 32, _q % 32
    _OUTPERM[_q] = _v * 32 + (2 * _r if _r < 16 else 2 * (_r - 16) + 1)

_VALPERM = np.concatenate([np.arange(0, 256, 2), np.arange(1, 256, 2)]).astype(np.int32)


def _matmul_body(x_ref, w_ref, b_ref, o_ref):
    o_ref[...] = (
        jnp.dot(x_ref[...], w_ref[...], preferred_element_type=jnp.float32)
        + b_ref[...]
    )


def _matmul(x, w_t, b, blk=512):
    n, k = x.shape
    m = w_t.shape[1]
    return pl.pallas_call(
        _matmul_body,
        grid=(n // blk,),
        in_specs=[
            pl.BlockSpec((blk, k), lambda i: (i, 0)),
            pl.BlockSpec((k, m), lambda i: (0, 0)),
            pl.BlockSpec((1, m), lambda i: (0, 0)),
        ],
        out_specs=pl.BlockSpec((blk, m), lambda i: (i, 0)),
        out_shape=jax.ShapeDtypeStruct((n, m), jnp.float32),
    )(x, w_t, b)


_NBLK_ACT = N_ACT // 512
_NBLK_TOT = TOT // 512


def _fused_body(act_ref, pas_ref, ids_ref, se_ref, vwt_ref, vb_ref,
                awt_ref, ab_ref, pa_ref, ps_ref, ah_ref,
                vals_ref, offs_ref, fw_ref):
    pid = pl.program_id(0)

    def pack_vals(x):
        vp = jnp.dot(x, vwt_ref[...], preferred_element_type=jnp.float32) + vb_ref[...]
        be = lax.bitcast_convert_type(vp[:, :128].astype(jnp.bfloat16), jnp.int16)
        bo = lax.bitcast_convert_type(vp[:, 128:].astype(jnp.bfloat16), jnp.int16)
        return (bo.astype(jnp.int32) << 16) | (be.astype(jnp.int32) & 0xFFFF)

    @pl.when(pid < _NBLK_ACT)
    def _():
        x = act_ref[...]
        vals_ref[...] = pack_vals(x)
        ids = ids_ref[...]
        r = x.shape[0]
        b = ids[:, 0:1]
        g = ids[:, 1:2]
        gx = ids[:, 2:3]
        gy = ids[:, 3:4]

        gb = jnp.broadcast_to(g, (r, FEAT))
        se = se_ref[...]
        af = x
        for mm in range(MAPS):
            af = af + jnp.where(
                gb == mm, jnp.broadcast_to(se[mm : mm + 1, :], (r, FEAT)), 0.0
            )

        l = jnp.dot(af, awt_ref[...], preferred_element_type=jnp.float32) + ab_ref[...]
        e = jnp.exp(l - jnp.max(l, axis=1, keepdims=True))
        ssum = jnp.dot(e, ah_ref[...], preferred_element_type=jnp.float32,
                       precision=lax.Precision.HIGHEST)
        aw = e / ssum

        lane = lax.broadcasted_iota(jnp.int32, (r, 128), 1)
        msl = lane % 16
        mj = msl // 4
        sj = msl % 4
        dx = sj % 2
        dy = sj // 2

        def selm(idx, vals):
            out = jnp.full((r, 128), vals[MAPS - 1], jnp.int32)
            for mm in range(MAPS - 1):
                out = jnp.where(idx == mm, vals[mm], out)
            return out

        smi = selm(mj, _S)
        base = selm(mj, _BASES)
        hw = smi * smi
        g128 = jnp.broadcast_to(g, (r, 128))
        sg = selm(g128, _S)
        sgf = sg.astype(jnp.float32)
        smf = smi.astype(jnp.float32)
        gxf = jnp.broadcast_to(gx, (r, 128)).astype(jnp.float32)
        gyf = jnp.broadcast_to(gy, (r, 128)).astype(jnp.float32)
        sx = (gxf + 0.5) / sgf * smf - 0.5
        sy = (gyf + 0.5) / sgf * smf - 0.5
        x0 = jnp.floor(sx).astype(jnp.int32)
        y0 = jnp.floor(sy).astype(jnp.int32)
        xi = x0 + dx
        yi = y0 + dy
        wx = 1.0 - jnp.abs(sx - xi.astype(jnp.float32))
        wy = 1.0 - jnp.abs(sy - yi.astype(jnp.float32))
        sw = wx * wy
        xc = jnp.clip(xi, 0, smi - 1)
        yc = jnp.clip(yi, 0, smi - 1)
        offs = base + jnp.broadcast_to(b, (r, 128)) * hw + yc * smi + xc
        offs_ref[...] = offs[:, :16]

        aw_p = jnp.dot(aw, pa_ref[...], preferred_element_type=jnp.float32,
                       precision=lax.Precision.HIGHEST)
        sw_p = jnp.dot(sw, ps_ref[...], preferred_element_type=jnp.float32,
                       precision=lax.Precision.HIGHEST)
        fw_ref[...] = aw_p * sw_p

    @pl.when(pid >= _NBLK_ACT)
    def _():
        vals_ref[...] = pack_vals(pas_ref[...])


def _fused_tc(act, pas, ids, se, vwt, vb, awt, ab):
    return pl.pallas_call(
        _fused_body,
        grid=(_NBLK_TOT,),
        in_specs=[
            pl.BlockSpec((512, FEAT), lambda i: (jnp.minimum(i, _NBLK_ACT - 1), 0)),
            pl.BlockSpec((512, FEAT), lambda i: (jnp.maximum(i - _NBLK_ACT, 0), 0)),
            pl.BlockSpec((512, 4), lambda i: (jnp.minimum(i, _NBLK_ACT - 1), 0)),
            pl.BlockSpec((MAPS, FEAT), lambda i: (0, 0)),
            pl.BlockSpec((FEAT, FEAT), lambda i: (0, 0)),
            pl.BlockSpec((1, FEAT), lambda i: (0, 0)),
            pl.BlockSpec((FEAT, 128), lambda i: (0, 0)),
            pl.BlockSpec((1, 128), lambda i: (0, 0)),
            pl.BlockSpec((128, 128), lambda i: (0, 0)),
            pl.BlockSpec((128, 128), lambda i: (0, 0)),
            pl.BlockSpec((128, 128), lambda i: (0, 0)),
        ],
        out_specs=[
            pl.BlockSpec((512, FEAT // 2), lambda i: (i, 0)),
            pl.BlockSpec((512, 16), lambda i: (jnp.minimum(i, _NBLK_ACT - 1), 0)),
            pl.BlockSpec((512, 128), lambda i: (jnp.minimum(i, _NBLK_ACT - 1), 0)),
        ],
        out_shape=[
            jax.ShapeDtypeStruct((TOT, FEAT // 2), jnp.int32),
            jax.ShapeDtypeStruct((N_ACT, 16), jnp.int32),
            jax.ShapeDtypeStruct((N_ACT, 128), jnp.float32),
        ],
    )(act, pas, ids, se, vwt, vb, awt, ab, _P_A, _P_S, _A_H)


def _sc_compute_chunk(rows_v, fw_v, out_v, out_hbm, fb):
    for f in range(_C):
        def ms_body(msp, acc):
            accl = list(acc)
            fwvec = fw_v[f, pl.ds(msp * 16, 16)]
            for ms_off in range(2):
                rr = f * 16 + msp * 2 + ms_off
                for h in range(HEADS):
                    w = fwvec[ms_off * 8 + h]
                    xi = rows_v[rr, pl.ds(h * 16, 16)]
                    xe = plsc.bitcast(xi << 16, jnp.float32)
                    xo = plsc.bitcast(xi & jnp.int32(-65536), jnp.float32)
                    accl[2 * h] = accl[2 * h] + w * xe
                    accl[2 * h + 1] = accl[2 * h + 1] + w * xo
            return tuple(accl)

        acc = lax.fori_loop(
            0, 8, ms_body,
            tuple(jnp.zeros((16,), jnp.float32) for _ in range(16)),
        )
        for v in range(16):
            out_v[f, pl.ds(v * 16, 16)] = acc[v]
    pltpu.sync_copy(out_v, out_hbm.at[pl.ds(fb, _C)])


def _sc_body(vals_hbm, fw_hbm, offs_hbm, idmap_hbm, out_hbm,
             idmap_v, offs_v, fid_v, fw_va, fw_vb, rows_va, rows_vb, out_v,
             sem_ra, sem_rb, sem_fa, sem_fb):
    wid = lax.axis_index("s") * _NC + lax.axis_index("c")
    base = wid * _RPT
    pltpu.sync_copy(idmap_hbm, idmap_v)
    pltpu.sync_copy(offs_hbm.at[pl.ds(base * 16, _RPT * 16)], offs_v)

    def fid_loop(f, c):
        fid = plsc.load_gather(idmap_v, [offs_v[pl.ds(f * 16, 16)]])
        fid_v[pl.ds(f * 16, 16)] = fid
        return c

    lax.fori_loop(0, _RPT, fid_loop, 0)

    def start(k, rows_v, fw_v, sem_r, sem_f):
        pltpu.async_copy(
            vals_hbm.at[fid_v.at[pl.ds(k * (_C * 16), _C * 16)]], rows_v, sem_r
        )
        pltpu.async_copy(fw_hbm.at[pl.ds(base + k * _C, _C)], fw_v, sem_f)

    def wait(k, rows_v, fw_v, sem_r, sem_f):
        pltpu.make_async_copy(
            vals_hbm.at[fid_v.at[pl.ds(k * (_C * 16), _C * 16)]], rows_v, sem_r
        ).wait()
        pltpu.make_async_copy(
            fw_hbm.at[pl.ds(base + k * _C, _C)], fw_v, sem_f
        ).wait()

    start(0, rows_va, fw_va, sem_ra, sem_fa)

    def pair(g, carry):
        ka = 2 * g
        kb = 2 * g + 1
        start(kb, rows_vb, fw_vb, sem_rb, sem_fb)
        wait(ka, rows_va, fw_va, sem_ra, sem_fa)
        _sc_compute_chunk(rows_va, fw_va, out_v, out_hbm, base + ka * _C)

        @pl.when(g < _NCH // 2 - 1)
        def _():
            start(ka + 2, rows_va, fw_va, sem_ra, sem_fa)

        wait(kb, rows_vb, fw_vb, sem_rb, sem_fb)
        _sc_compute_chunk(rows_vb, fw_vb, out_v, out_hbm, base + kb * _C)
        return carry

    lax.fori_loop(0, _NCH // 2, pair, 0)


def _sc_gather_combine(all_vals, fw, offs, flat_idmap):
    mesh = plsc.VectorSubcoreMesh(
        core_axis_name="c", subcore_axis_name="s", num_cores=_NC, num_subcores=_NS
    )
    run = functools.partial(
        pl.kernel,
        out_type=jax.ShapeDtypeStruct((N_ACT, FEAT), jnp.float32),
        mesh=mesh,
        compiler_params=pltpu.CompilerParams(needs_layout_passes=False),
        scratch_types=[
            pltpu.VMEM((_IDTOT,), jnp.int32),
            pltpu.VMEM((_RPT * 16,), jnp.int32),
            pltpu.VMEM((_RPT * 16,), jnp.int32),
            pltpu.VMEM((_C, 128), jnp.float32),
            pltpu.VMEM((_C, 128), jnp.float32),
            pltpu.VMEM((_C * 16, FEAT // 2), jnp.int32),
            pltpu.VMEM((_C * 16, FEAT // 2), jnp.int32),
            pltpu.VMEM((_C, FEAT), jnp.float32),
            pltpu.SemaphoreType.DMA,
            pltpu.SemaphoreType.DMA,
            pltpu.SemaphoreType.DMA,
            pltpu.SemaphoreType.DMA,
        ],
    )(_sc_body)
    return run(all_vals, fw, offs, flat_idmap)


def kernel(in_act_feats, act_batch_ids, act_map_ids, act_xy_ids, map_shapes,
           pas_feats, id_map0, id_map1, id_map2, id_map3, scale_embed,
           attn_W, attn_b, val_W, val_b, out_W, out_b):
    del map_shapes
    ids = jnp.stack(
        [act_batch_ids, act_map_ids, act_xy_ids[:, 0], act_xy_ids[:, 1]], axis=1
    )
    attn_wr_t = attn_W.reshape(HEADS, MAPS, FEAT).transpose(1, 0, 2).reshape(
        HEADS * MAPS, FEAT).T
    awt128 = jnp.concatenate([attn_wr_t] * 4, axis=1)
    attn_br = attn_b.reshape(HEADS, MAPS).T.reshape(1, HEADS * MAPS)
    ab128 = jnp.concatenate([attn_br] * 4, axis=1)
    flat_idmap = jnp.concatenate(
        [m.reshape(-1) for m in (id_map0, id_map1, id_map2, id_map3)]
    )
    vwt_perm = val_W.T.reshape(FEAT, 128, 2).transpose(0, 2, 1).reshape(FEAT, FEAT)
    vb_perm = val_b.reshape(128, 2).transpose(1, 0).reshape(1, FEAT)
    vals_i32, offs, fw = _fused_tc(
        in_act_feats, pas_feats, ids, scale_embed, vwt_perm, vb_perm, awt128, ab128,
    )
    val_feats = _sc_gather_combine(vals_i32, fw, offs.reshape(-1), flat_idmap)
    out_wt_perm = out_W.T.reshape(8, 16, 2, FEAT).transpose(0, 2, 1, 3).reshape(
        FEAT, FEAT)
    return _matmul(val_feats, out_wt_perm, out_b.reshape(1, FEAT))


# speedup vs baseline: 1.5400x; 1.0075x over previous
import functools

import jax
import jax.numpy as jnp
import numpy as np
from jax import lax
from jax.experimental import pallas as pl
from jax.experimental.pallas import tpu as pltpu
from jax.experimental.pallas import tpu_sc as plsc

N_ACT = 16384
N_PAS = 8192
TOT = N_ACT + N_PAS
FEAT = 256
HEADS = 8
MAPS = 4
BATCH = 4
_S = (64, 32, 16, 8)
_BASES = (0, 16384, 20480, 21504)
_IDTOT = 21760

_NC, _NS = 2, 16
_NW = _NC * _NS
_C = 8
_RPT = N_ACT // _NW
_NCH = _RPT // _C

_LANE = np.arange(128)
_MSL = _LANE % 16
_MJ = _MSL // 4
_SJ = _MSL % 4
_SM128 = np.asarray(_S, np.int32)[_MJ]
_BASE128 = np.asarray(_BASES, np.int32)[_MJ]
_HW128 = (_SM128.astype(np.int64) ** 2).astype(np.int32)
_DX128 = (_SJ % 2).astype(np.int32)
_DY128 = (_SJ // 2).astype(np.int32)

_P_A = np.zeros((128, 128), np.float32)
for _j in range(128):
    _P_A[8 * (_j // 32) + (_j % 8), _j] = 1.0
_P_S = np.zeros((128, 128), np.float32)
for _j in range(128):
    _P_S[4 * (_j // 32) + ((_j // 8) % 4), _j] = 1.0
_A_H = np.zeros((128, 128), np.float32)
for _j in range(128):
    for _l in range(32):
        if _l % 8 == _j % 8:
            _A_H[_l, _j] = 1.0

_OUTPERM = np.empty((256,), np.int32)
for _q in range(256):
    _v, _r = _q // 32, _q % 32
    _OUTPERM[_q] = _v * 32 + (2 * _r if _r < 16 else 2 * (_r - 16) + 1)

_VALPERM = np.concatenate([np.arange(0, 256, 2), np.arange(1, 256, 2)]).astype(np.int32)


def _matmul_body(x_ref, w_ref, b_ref, o_ref):
    o_ref[...] = (
        jnp.dot(x_ref[...], w_ref[...], preferred_element_type=jnp.float32)
        + b_ref[...]
    )


def _matmul(x, w_t, b, blk=512):
    n, k = x.shape
    m = w_t.shape[1]
    return pl.pallas_call(
        _matmul_body,
        grid=(n // blk,),
        in_specs=[
            pl.BlockSpec((blk, k), lambda i: (i, 0)),
            pl.BlockSpec((k, m), lambda i: (0, 0)),
            pl.BlockSpec((1, m), lambda i: (0, 0)),
        ],
        out_specs=pl.BlockSpec((blk, m), lambda i: (i, 0)),
        out_shape=jax.ShapeDtypeStruct((n, m), jnp.float32),
    )(x, w_t, b)


_NBLK_ACT = N_ACT // 512
_NBLK_TOT = TOT // 512


def _fused_body(act_ref, pas_ref, ids_ref, se_ref, vwt_ref, vb_ref,
                awt_ref, ab_ref, pa_ref, ps_ref, ah_ref,
                vals_ref, offs_ref, fw_ref):
    pid = pl.program_id(0)

    def pack_vals(x):
        vp = jnp.dot(x, vwt_ref[...], preferred_element_type=jnp.float32) + vb_ref[...]
        be = lax.bitcast_convert_type(vp[:, :128].astype(jnp.bfloat16), jnp.int16)
        bo = lax.bitcast_convert_type(vp[:, 128:].astype(jnp.bfloat16), jnp.int16)
        return (bo.astype(jnp.int32) << 16) | (be.astype(jnp.int32) & 0xFFFF)

    @pl.when(pid < _NBLK_ACT)
    def _():
        x = act_ref[...]
        vals_ref[...] = pack_vals(x)
        ids = ids_ref[...]
        r = x.shape[0]
        b = ids[:, 0:1]
        g = ids[:, 1:2]
        gx = ids[:, 2:3]
        gy = ids[:, 3:4]

        gb = jnp.broadcast_to(g, (r, FEAT))
        se = se_ref[...]
        af = x
        for mm in range(MAPS):
            af = af + jnp.where(
                gb == mm, jnp.broadcast_to(se[mm : mm + 1, :], (r, FEAT)), 0.0
            )

        l = jnp.dot(af, awt_ref[...], preferred_element_type=jnp.float32) + ab_ref[...]
        e = jnp.exp(l - jnp.max(l, axis=1, keepdims=True))
        ssum = jnp.dot(e, ah_ref[...], preferred_element_type=jnp.float32,
                       precision=lax.Precision.HIGHEST)
        aw = e / ssum

        lane = lax.broadcasted_iota(jnp.int32, (r, 128), 1)
        msl = lane % 16
        mj = msl // 4
        sj = msl % 4
        dx = sj % 2
        dy = sj // 2

        def selm(idx, vals):
            out = jnp.full((r, 128), vals[MAPS - 1], jnp.int32)
            for mm in range(MAPS - 1):
                out = jnp.where(idx == mm, vals[mm], out)
            return out

        smi = selm(mj, _S)
        base = selm(mj, _BASES)
        hw = smi * smi
        g128 = jnp.broadcast_to(g, (r, 128))
        sg = selm(g128, _S)
        sgf = sg.astype(jnp.float32)
        smf = smi.astype(jnp.float32)
        gxf = jnp.broadcast_to(gx, (r, 128)).astype(jnp.float32)
        gyf = jnp.broadcast_to(gy, (r, 128)).astype(jnp.float32)
        sx = (gxf + 0.5) / sgf * smf - 0.5
        sy = (gyf + 0.5) / sgf * smf - 0.5
        x0 = jnp.floor(sx).astype(jnp.int32)
        y0 = jnp.floor(sy).astype(jnp.int32)
        xi = x0 + dx
        yi = y0 + dy
        wx = 1.0 - jnp.abs(sx - xi.astype(jnp.float32))
        wy = 1.0 - jnp.abs(sy - yi.astype(jnp.float32))
        sw = wx * wy
        xc = jnp.clip(xi, 0, smi - 1)
        yc = jnp.clip(yi, 0, smi - 1)
        offs = base + jnp.broadcast_to(b, (r, 128)) * hw + yc * smi + xc
        offs_ref[...] = offs[:, :16]

        aw_p = jnp.dot(aw, pa_ref[...], preferred_element_type=jnp.float32,
                       precision=lax.Precision.HIGHEST)
        sw_p = jnp.dot(sw, ps_ref[...], preferred_element_type=jnp.float32,
                       precision=lax.Precision.HIGHEST)
        fw_ref[...] = aw_p * sw_p

    @pl.when(pid >= _NBLK_ACT)
    def _():
        vals_ref[...] = pack_vals(pas_ref[...])


def _fused_tc(act, pas, ids, se, vwt, vb, awt, ab):
    return pl.pallas_call(
        _fused_body,
        grid=(_NBLK_TOT,),
        in_specs=[
            pl.BlockSpec((512, FEAT), lambda i: (jnp.minimum(i, _NBLK_ACT - 1), 0)),
            pl.BlockSpec((512, FEAT), lambda i: (jnp.maximum(i - _NBLK_ACT, 0), 0)),
            pl.BlockSpec((512, 4), lambda i: (jnp.minimum(i, _NBLK_ACT - 1), 0)),
            pl.BlockSpec((MAPS, FEAT), lambda i: (0, 0)),
            pl.BlockSpec((FEAT, FEAT), lambda i: (0, 0)),
            pl.BlockSpec((1, FEAT), lambda i: (0, 0)),
            pl.BlockSpec((FEAT, 128), lambda i: (0, 0)),
            pl.BlockSpec((1, 128), lambda i: (0, 0)),
            pl.BlockSpec((128, 128), lambda i: (0, 0)),
            pl.BlockSpec((128, 128), lambda i: (0, 0)),
            pl.BlockSpec((128, 128), lambda i: (0, 0)),
        ],
        out_specs=[
            pl.BlockSpec((512, FEAT // 2), lambda i: (i, 0)),
            pl.BlockSpec((512, 16), lambda i: (jnp.minimum(i, _NBLK_ACT - 1), 0)),
            pl.BlockSpec((512, 128), lambda i: (jnp.minimum(i, _NBLK_ACT - 1), 0)),
        ],
        out_shape=[
            jax.ShapeDtypeStruct((TOT, FEAT // 2), jnp.int32),
            jax.ShapeDtypeStruct((N_ACT, 16), jnp.int32),
            jax.ShapeDtypeStruct((N_ACT, 128), jnp.float32),
        ],
    )(act, pas, ids, se, vwt, vb, awt, ab, _P_A, _P_S, _A_H)


def _sc_compute_chunk(rows_v, fw_v, out_v, out_hbm, fb, sem_o):
    for f in range(_C):
        def ms_body(msp, acc):
            accl = list(acc)
            fwvec = fw_v[f, pl.ds(msp * 16, 16)]
            for ms_off in range(2):
                rr = f * 16 + msp * 2 + ms_off
                for h in range(HEADS):
                    w = fwvec[ms_off * 8 + h]
                    xi = rows_v[rr, pl.ds(h * 16, 16)]
                    xe = plsc.bitcast(xi << 16, jnp.float32)
                    xo = plsc.bitcast(xi & jnp.int32(-65536), jnp.float32)
                    accl[2 * h] = accl[2 * h] + w * xe
                    accl[2 * h + 1] = accl[2 * h + 1] + w * xo
            return tuple(accl)

        acc = lax.fori_loop(
            0, 8, ms_body,
            tuple(jnp.zeros((16,), jnp.float32) for _ in range(16)),
        )
        for v in range(16):
            out_v[f, pl.ds(v * 16, 16)] = acc[v]
    pltpu.async_copy(out_v, out_hbm.at[pl.ds(fb, _C)], sem_o)


def _sc_body(vals_hbm, fw_hbm, offs_hbm, idmap_hbm, out_hbm,
             idmap_v, offs_v, fid_v, fw_va, fw_vb, rows_va, rows_vb,
             out_va, out_vb,
             sem_ra, sem_rb, sem_fa, sem_fb, sem_oa, sem_ob):
    wid = lax.axis_index("s") * _NC + lax.axis_index("c")
    base = wid * _RPT
    pltpu.sync_copy(idmap_hbm, idmap_v)
    pltpu.sync_copy(offs_hbm.at[pl.ds(base * 16, _RPT * 16)], offs_v)

    def fid_loop(f, c):
        fid = plsc.load_gather(idmap_v, [offs_v[pl.ds(f * 16, 16)]])
        fid_v[pl.ds(f * 16, 16)] = fid
        return c

    lax.fori_loop(0, _RPT, fid_loop, 0)

    def start(k, rows_v, fw_v, sem_r, sem_f):
        pltpu.async_copy(
            vals_hbm.at[fid_v.at[pl.ds(k * (_C * 16), _C * 16)]], rows_v, sem_r
        )
        pltpu.async_copy(fw_hbm.at[pl.ds(base + k * _C, _C)], fw_v, sem_f)

    def wait(k, rows_v, fw_v, sem_r, sem_f):
        pltpu.make_async_copy(
            vals_hbm.at[fid_v.at[pl.ds(k * (_C * 16), _C * 16)]], rows_v, sem_r
        ).wait()
        pltpu.make_async_copy(
            fw_hbm.at[pl.ds(base + k * _C, _C)], fw_v, sem_f
        ).wait()

    def wait_out(k, out_v, sem_o):
        pltpu.make_async_copy(
            out_v, out_hbm.at[pl.ds(base + k * _C, _C)], sem_o
        ).wait()

    start(0, rows_va, fw_va, sem_ra, sem_fa)

    def pair(g, carry):
        ka = 2 * g
        kb = 2 * g + 1
        start(kb, rows_vb, fw_vb, sem_rb, sem_fb)
        wait(ka, rows_va, fw_va, sem_ra, sem_fa)

        @pl.when(g > 0)
        def _():
            wait_out(ka - 2, out_va, sem_oa)

        _sc_compute_chunk(rows_va, fw_va, out_va, out_hbm, base + ka * _C, sem_oa)

        @pl.when(g < _NCH // 2 - 1)
        def _():
            start(ka + 2, rows_va, fw_va, sem_ra, sem_fa)

        wait(kb, rows_vb, fw_vb, sem_rb, sem_fb)

        @pl.when(g > 0)
        def _():
            wait_out(kb - 2, out_vb, sem_ob)

        _sc_compute_chunk(rows_vb, fw_vb, out_vb, out_hbm, base + kb * _C, sem_ob)
        return carry

    lax.fori_loop(0, _NCH // 2, pair, 0)
    wait_out(_NCH - 2, out_va, sem_oa)
    wait_out(_NCH - 1, out_vb, sem_ob)


def _sc_gather_combine(all_vals, fw, offs, flat_idmap):
    mesh = plsc.VectorSubcoreMesh(
        core_axis_name="c", subcore_axis_name="s", num_cores=_NC, num_subcores=_NS
    )
    run = functools.partial(
        pl.kernel,
        out_type=jax.ShapeDtypeStruct((N_ACT, FEAT), jnp.float32),
        mesh=mesh,
        compiler_params=pltpu.CompilerParams(needs_layout_passes=False),
        scratch_types=[
            pltpu.VMEM((_IDTOT,), jnp.int32),
            pltpu.VMEM((_RPT * 16,), jnp.int32),
            pltpu.VMEM((_RPT * 16,), jnp.int32),
            pltpu.VMEM((_C, 128), jnp.float32),
            pltpu.VMEM((_C, 128), jnp.float32),
            pltpu.VMEM((_C * 16, FEAT // 2), jnp.int32),
            pltpu.VMEM((_C * 16, FEAT // 2), jnp.int32),
            pltpu.VMEM((_C, FEAT), jnp.float32),
            pltpu.VMEM((_C, FEAT), jnp.float32),
            pltpu.SemaphoreType.DMA,
            pltpu.SemaphoreType.DMA,
            pltpu.SemaphoreType.DMA,
            pltpu.SemaphoreType.DMA,
            pltpu.SemaphoreType.DMA,
            pltpu.SemaphoreType.DMA,
        ],
    )(_sc_body)
    return run(all_vals, fw, offs, flat_idmap)


def kernel(in_act_feats, act_batch_ids, act_map_ids, act_xy_ids, map_shapes,
           pas_feats, id_map0, id_map1, id_map2, id_map3, scale_embed,
           attn_W, attn_b, val_W, val_b, out_W, out_b):
    del map_shapes
    ids = jnp.stack(
        [act_batch_ids, act_map_ids, act_xy_ids[:, 0], act_xy_ids[:, 1]], axis=1
    )
    attn_wr_t = attn_W.reshape(HEADS, MAPS, FEAT).transpose(1, 0, 2).reshape(
        HEADS * MAPS, FEAT).T
    awt128 = jnp.concatenate([attn_wr_t] * 4, axis=1)
    attn_br = attn_b.reshape(HEADS, MAPS).T.reshape(1, HEADS * MAPS)
    ab128 = jnp.concatenate([attn_br] * 4, axis=1)
    flat_idmap = jnp.concatenate(
        [m.reshape(-1) for m in (id_map0, id_map1, id_map2, id_map3)]
    )
    vwt_perm = val_W.T.reshape(FEAT, 128, 2).transpose(0, 2, 1).reshape(FEAT, FEAT)
    vb_perm = val_b.reshape(128, 2).transpose(1, 0).reshape(1, FEAT)
    vals_i32, offs, fw = _fused_tc(
        in_act_feats, pas_feats, ids, scale_embed, vwt_perm, vb_perm, awt128, ab128,
    )
    val_feats = _sc_gather_combine(vals_i32, fw, offs.reshape(-1), flat_idmap)
    out_wt_perm = out_W.T.reshape(8, 16, 2, FEAT).transpose(0, 2, 1, 3).reshape(
        FEAT, FEAT)
    return _matmul(val_feats, out_wt_perm, out_b.reshape(1, FEAT))


# speedup vs baseline: 1.6731x; 1.0864x over previous
import functools

import jax
import jax.numpy as jnp
import numpy as np
from jax import lax
from jax.experimental import pallas as pl
from jax.experimental.pallas import tpu as pltpu
from jax.experimental.pallas import tpu_sc as plsc

N_ACT = 16384
N_PAS = 8192
TOT = N_ACT + N_PAS
FEAT = 256
HEADS = 8
MAPS = 4
BATCH = 4
_S = (64, 32, 16, 8)
_BASES = (0, 16384, 20480, 21504)
_IDTOT = 21760

_NC, _NS = 2, 16
_NW = _NC * _NS
_C = 8
_RPT = N_ACT // _NW
_NCH = _RPT // _C

_LANE = np.arange(128)
_MSL = _LANE % 16
_MJ = _MSL // 4
_SJ = _MSL % 4
_SM128 = np.asarray(_S, np.int32)[_MJ]
_BASE128 = np.asarray(_BASES, np.int32)[_MJ]
_HW128 = (_SM128.astype(np.int64) ** 2).astype(np.int32)
_DX128 = (_SJ % 2).astype(np.int32)
_DY128 = (_SJ // 2).astype(np.int32)

_P_A = np.zeros((128, 128), np.float32)
for _j in range(128):
    _P_A[8 * (_j // 32) + (_j % 8), _j] = 1.0
_P_S = np.zeros((128, 128), np.float32)
for _j in range(128):
    _P_S[4 * (_j // 32) + ((_j // 8) % 4), _j] = 1.0
_A_H = np.zeros((128, 128), np.float32)
for _j in range(128):
    for _l in range(32):
        if _l % 8 == _j % 8:
            _A_H[_l, _j] = 1.0

_OUTPERM = np.empty((256,), np.int32)
for _q in range(256):
    _v, _r = _q // 32, _q % 32
    _OUTPERM[_q] = _v * 32 + (2 * _r if _r < 16 else 2 * (_r - 16) + 1)

_VALPERM = np.concatenate([np.arange(0, 256, 2), np.arange(1, 256, 2)]).astype(np.int32)


def _matmul_body(x_ref, w_ref, b_ref, o_ref):
    o_ref[...] = (
        jnp.dot(x_ref[...], w_ref[...], preferred_element_type=jnp.float32)
        + b_ref[...]
    )


def _matmul(x, w_t, b, blk=512):
    n, k = x.shape
    m = w_t.shape[1]
    return pl.pallas_call(
        _matmul_body,
        grid=(n // blk,),
        in_specs=[
            pl.BlockSpec((blk, k), lambda i: (i, 0)),
            pl.BlockSpec((k, m), lambda i: (0, 0)),
            pl.BlockSpec((1, m), lambda i: (0, 0)),
        ],
        out_specs=pl.BlockSpec((blk, m), lambda i: (i, 0)),
        out_shape=jax.ShapeDtypeStruct((n, m), jnp.float32),
    )(x, w_t, b)


_NBLK_ACT = N_ACT // 512
_NBLK_TOT = TOT // 512


def _fused_body(act_ref, pas_ref, ids_ref, se_ref, vwt_ref, vb_ref,
                awt_ref, ab_ref, pa_ref, ps_ref, ah_ref,
                vals_ref, offs_ref, fw_ref):
    pid = pl.program_id(0)

    def pack_vals(x):
        vp = jnp.dot(x, vwt_ref[...], preferred_element_type=jnp.float32) + vb_ref[...]
        be = lax.bitcast_convert_type(vp[:, :128].astype(jnp.bfloat16), jnp.int16)
        bo = lax.bitcast_convert_type(vp[:, 128:].astype(jnp.bfloat16), jnp.int16)
        return (bo.astype(jnp.int32) << 16) | (be.astype(jnp.int32) & 0xFFFF)

    @pl.when(pid < _NBLK_ACT)
    def _():
        x = act_ref[...]
        vals_ref[...] = pack_vals(x)
        ids = ids_ref[...]
        r = x.shape[0]
        b = ids[:, 0:1]
        g = ids[:, 1:2]
        gx = ids[:, 2:3]
        gy = ids[:, 3:4]

        gb = jnp.broadcast_to(g, (r, FEAT))
        se = se_ref[...]
        af = x
        for mm in range(MAPS):
            af = af + jnp.where(
                gb == mm, jnp.broadcast_to(se[mm : mm + 1, :], (r, FEAT)), 0.0
            )

        l = jnp.dot(af, awt_ref[...], preferred_element_type=jnp.float32) + ab_ref[...]
        e = jnp.exp(l - jnp.max(l, axis=1, keepdims=True))
        ssum = jnp.dot(e, ah_ref[...], preferred_element_type=jnp.float32,
                       precision=lax.Precision.HIGHEST)
        aw = e / ssum

        lane = lax.broadcasted_iota(jnp.int32, (r, 128), 1)
        msl = lane % 16
        mj = msl // 4
        sj = msl % 4
        dx = sj % 2
        dy = sj // 2

        def selm(idx, vals):
            out = jnp.full((r, 128), vals[MAPS - 1], jnp.int32)
            for mm in range(MAPS - 1):
                out = jnp.where(idx == mm, vals[mm], out)
            return out

        smi = selm(mj, _S)
        base = selm(mj, _BASES)
        hw = smi * smi
        g128 = jnp.broadcast_to(g, (r, 128))
        sg = selm(g128, _S)
        sgf = sg.astype(jnp.float32)
        smf = smi.astype(jnp.float32)
        gxf = jnp.broadcast_to(gx, (r, 128)).astype(jnp.float32)
        gyf = jnp.broadcast_to(gy, (r, 128)).astype(jnp.float32)
        sx = (gxf + 0.5) / sgf * smf - 0.5
        sy = (gyf + 0.5) / sgf * smf - 0.5
        x0 = jnp.floor(sx).astype(jnp.int32)
        y0 = jnp.floor(sy).astype(jnp.int32)
        xi = x0 + dx
        yi = y0 + dy
        wx = 1.0 - jnp.abs(sx - xi.astype(jnp.float32))
        wy = 1.0 - jnp.abs(sy - yi.astype(jnp.float32))
        sw = wx * wy
        xc = jnp.clip(xi, 0, smi - 1)
        yc = jnp.clip(yi, 0, smi - 1)
        offs = base + jnp.broadcast_to(b, (r, 128)) * hw + yc * smi + xc
        offs_ref[...] = offs[:, :16]

        aw_p = jnp.dot(aw, pa_ref[...], preferred_element_type=jnp.float32,
                       precision=lax.Precision.HIGHEST)
        sw_p = jnp.dot(sw, ps_ref[...], preferred_element_type=jnp.float32,
                       precision=lax.Precision.HIGHEST)
        fw_ref[...] = aw_p * sw_p

    @pl.when(pid >= _NBLK_ACT)
    def _():
        vals_ref[...] = pack_vals(pas_ref[...])


def _fused_tc(act, pas, ids, se, vwt, vb, awt, ab):
    return pl.pallas_call(
        _fused_body,
        grid=(_NBLK_TOT,),
        in_specs=[
            pl.BlockSpec((512, FEAT), lambda i: (jnp.minimum(i, _NBLK_ACT - 1), 0)),
            pl.BlockSpec((512, FEAT), lambda i: (jnp.maximum(i - _NBLK_ACT, 0), 0)),
            pl.BlockSpec((512, 4), lambda i: (jnp.minimum(i, _NBLK_ACT - 1), 0)),
            pl.BlockSpec((MAPS, FEAT), lambda i: (0, 0)),
            pl.BlockSpec((FEAT, FEAT), lambda i: (0, 0)),
            pl.BlockSpec((1, FEAT), lambda i: (0, 0)),
            pl.BlockSpec((FEAT, 128), lambda i: (0, 0)),
            pl.BlockSpec((1, 128), lambda i: (0, 0)),
            pl.BlockSpec((128, 128), lambda i: (0, 0)),
            pl.BlockSpec((128, 128), lambda i: (0, 0)),
            pl.BlockSpec((128, 128), lambda i: (0, 0)),
        ],
        out_specs=[
            pl.BlockSpec((512, FEAT // 2), lambda i: (i, 0)),
            pl.BlockSpec((512, 16), lambda i: (jnp.minimum(i, _NBLK_ACT - 1), 0)),
            pl.BlockSpec((512, 128), lambda i: (jnp.minimum(i, _NBLK_ACT - 1), 0)),
        ],
        out_shape=[
            jax.ShapeDtypeStruct((TOT, FEAT // 2), jnp.int32),
            jax.ShapeDtypeStruct((N_ACT, 16), jnp.int32),
            jax.ShapeDtypeStruct((N_ACT, 128), jnp.float32),
        ],
    )(act, pas, ids, se, vwt, vb, awt, ab, _P_A, _P_S, _A_H)


def _sc_compute_chunk(rows_v, fw_v, out_v, out_hbm, fb, sem_o):
    for f in range(_C):
        def ms_body(msp, acc):
            accl = list(acc)
            fwvec = fw_v[f, pl.ds(msp * 16, 16)]
            for ms_off in range(2):
                rr = f * 16 + msp * 2 + ms_off
                for h in range(HEADS):
                    w = fwvec[ms_off * 8 + h]
                    xi = rows_v[rr, pl.ds(h * 16, 16)]
                    xe = plsc.bitcast(xi << 16, jnp.float32)
                    xo = plsc.bitcast(xi, jnp.float32)
                    accl[2 * h] = accl[2 * h] + w * xe
                    accl[2 * h + 1] = accl[2 * h + 1] + w * xo
            return tuple(accl)

        acc = lax.fori_loop(
            0, 8, ms_body,
            tuple(jnp.zeros((16,), jnp.float32) for _ in range(16)),
        )
        for v in range(16):
            out_v[f, pl.ds(v * 16, 16)] = acc[v]
    pltpu.async_copy(out_v, out_hbm.at[pl.ds(fb, _C)], sem_o)


def _sc_body(vals_hbm, fw_hbm, offs_hbm, idmap_hbm, out_hbm,
             idmap_v, offs_v, fid_v, fw_va, fw_vb, rows_va, rows_vb,
             out_va, out_vb,
             sem_ra, sem_rb, sem_fa, sem_fb, sem_oa, sem_ob):
    wid = lax.axis_index("s") * _NC + lax.axis_index("c")
    base = wid * _RPT
    pltpu.sync_copy(idmap_hbm, idmap_v)
    pltpu.sync_copy(offs_hbm.at[pl.ds(base * 16, _RPT * 16)], offs_v)

    def fid_loop(f, c):
        fid = plsc.load_gather(idmap_v, [offs_v[pl.ds(f * 16, 16)]])
        fid_v[pl.ds(f * 16, 16)] = fid
        return c

    lax.fori_loop(0, _RPT, fid_loop, 0)

    def start(k, rows_v, fw_v, sem_r, sem_f):
        pltpu.async_copy(
            vals_hbm.at[fid_v.at[pl.ds(k * (_C * 16), _C * 16)]], rows_v, sem_r
        )
        pltpu.async_copy(fw_hbm.at[pl.ds(base + k * _C, _C)], fw_v, sem_f)

    def wait(k, rows_v, fw_v, sem_r, sem_f):
        pltpu.make_async_copy(
            vals_hbm.at[fid_v.at[pl.ds(k * (_C * 16), _C * 16)]], rows_v, sem_r
        ).wait()
        pltpu.make_async_copy(
            fw_hbm.at[pl.ds(base + k * _C, _C)], fw_v, sem_f
        ).wait()

    def wait_out(k, out_v, sem_o):
        pltpu.make_async_copy(
            out_v, out_hbm.at[pl.ds(base + k * _C, _C)], sem_o
        ).wait()

    start(0, rows_va, fw_va, sem_ra, sem_fa)

    def pair(g, carry):
        ka = 2 * g
        kb = 2 * g + 1
        start(kb, rows_vb, fw_vb, sem_rb, sem_fb)
        wait(ka, rows_va, fw_va, sem_ra, sem_fa)

        @pl.when(g > 0)
        def _():
            wait_out(ka - 2, out_va, sem_oa)

        _sc_compute_chunk(rows_va, fw_va, out_va, out_hbm, base + ka * _C, sem_oa)

        @pl.when(g < _NCH // 2 - 1)
        def _():
            start(ka + 2, rows_va, fw_va, sem_ra, sem_fa)

        wait(kb, rows_vb, fw_vb, sem_rb, sem_fb)

        @pl.when(g > 0)
        def _():
            wait_out(kb - 2, out_vb, sem_ob)

        _sc_compute_chunk(rows_vb, fw_vb, out_vb, out_hbm, base + kb * _C, sem_ob)
        return carry

    lax.fori_loop(0, _NCH // 2, pair, 0)
    wait_out(_NCH - 2, out_va, sem_oa)
    wait_out(_NCH - 1, out_vb, sem_ob)


def _sc_gather_combine(all_vals, fw, offs, flat_idmap):
    mesh = plsc.VectorSubcoreMesh(
        core_axis_name="c", subcore_axis_name="s", num_cores=_NC, num_subcores=_NS
    )
    run = functools.partial(
        pl.kernel,
        out_type=jax.ShapeDtypeStruct((N_ACT, FEAT), jnp.float32),
        mesh=mesh,
        compiler_params=pltpu.CompilerParams(needs_layout_passes=False),
        scratch_types=[
            pltpu.VMEM((_IDTOT,), jnp.int32),
            pltpu.VMEM((_RPT * 16,), jnp.int32),
            pltpu.VMEM((_RPT * 16,), jnp.int32),
            pltpu.VMEM((_C, 128), jnp.float32),
            pltpu.VMEM((_C, 128), jnp.float32),
            pltpu.VMEM((_C * 16, FEAT // 2), jnp.int32),
            pltpu.VMEM((_C * 16, FEAT // 2), jnp.int32),
            pltpu.VMEM((_C, FEAT), jnp.float32),
            pltpu.VMEM((_C, FEAT), jnp.float32),
            pltpu.SemaphoreType.DMA,
            pltpu.SemaphoreType.DMA,
            pltpu.SemaphoreType.DMA,
            pltpu.SemaphoreType.DMA,
            pltpu.SemaphoreType.DMA,
            pltpu.SemaphoreType.DMA,
        ],
    )(_sc_body)
    return run(all_vals, fw, offs, flat_idmap)


def kernel(in_act_feats, act_batch_ids, act_map_ids, act_xy_ids, map_shapes,
           pas_feats, id_map0, id_map1, id_map2, id_map3, scale_embed,
           attn_W, attn_b, val_W, val_b, out_W, out_b):
    del map_shapes
    ids = jnp.stack(
        [act_batch_ids, act_map_ids, act_xy_ids[:, 0], act_xy_ids[:, 1]], axis=1
    )
    attn_wr_t = attn_W.reshape(HEADS, MAPS, FEAT).transpose(1, 0, 2).reshape(
        HEADS * MAPS, FEAT).T
    awt128 = jnp.concatenate([attn_wr_t] * 4, axis=1)
    attn_br = attn_b.reshape(HEADS, MAPS).T.reshape(1, HEADS * MAPS)
    ab128 = jnp.concatenate([attn_br] * 4, axis=1)
    flat_idmap = jnp.concatenate(
        [m.reshape(-1) for m in (id_map0, id_map1, id_map2, id_map3)]
    )
    vwt_perm = val_W.T.reshape(FEAT, 128, 2).transpose(0, 2, 1).reshape(FEAT, FEAT)
    vb_perm = val_b.reshape(128, 2).transpose(1, 0).reshape(1, FEAT)
    vals_i32, offs, fw = _fused_tc(
        in_act_feats, pas_feats, ids, scale_embed, vwt_perm, vb_perm, awt128, ab128,
    )
    val_feats = _sc_gather_combine(vals_i32, fw, offs.reshape(-1), flat_idmap)
    out_wt_perm = out_W.T.reshape(8, 16, 2, FEAT).transpose(0, 2, 1, 3).reshape(
        FEAT, FEAT)
    return _matmul(val_feats, out_wt_perm, out_b.reshape(1, FEAT))


# speedup vs baseline: 1.6924x; 1.0116x over previous
import functools

import jax
import jax.numpy as jnp
import numpy as np
from jax import lax
from jax.experimental import pallas as pl
from jax.experimental.pallas import tpu as pltpu
from jax.experimental.pallas import tpu_sc as plsc

N_ACT = 16384
N_PAS = 8192
TOT = N_ACT + N_PAS
FEAT = 256
HEADS = 8
MAPS = 4
BATCH = 4
_S = (64, 32, 16, 8)
_BASES = (0, 16384, 20480, 21504)
_IDTOT = 21760

_NC, _NS = 2, 16
_NW = _NC * _NS
_C = 8
_RPT = N_ACT // _NW
_NCH = _RPT // _C

_LANE = np.arange(128)
_MSL = _LANE % 16
_MJ = _MSL // 4
_SJ = _MSL % 4
_SM128 = np.asarray(_S, np.int32)[_MJ]
_BASE128 = np.asarray(_BASES, np.int32)[_MJ]
_HW128 = (_SM128.astype(np.int64) ** 2).astype(np.int32)
_DX128 = (_SJ % 2).astype(np.int32)
_DY128 = (_SJ // 2).astype(np.int32)

_P_A = np.zeros((128, 128), np.float32)
for _j in range(128):
    _P_A[8 * (_j // 32) + (_j % 8), _j] = 1.0
_P_S = np.zeros((128, 128), np.float32)
for _j in range(128):
    _P_S[4 * (_j // 32) + ((_j // 8) % 4), _j] = 1.0
_A_H = np.zeros((128, 128), np.float32)
for _j in range(128):
    for _l in range(32):
        if _l % 8 == _j % 8:
            _A_H[_l, _j] = 1.0

_OUTPERM = np.empty((256,), np.int32)
for _q in range(256):
    _v, _r = _q // 32, _q % 32
    _OUTPERM[_q] = _v * 32 + (2 * _r if _r < 16 else 2 * (_r - 16) + 1)

_VALPERM = np.concatenate([np.arange(0, 256, 2), np.arange(1, 256, 2)]).astype(np.int32)


def _matmul_body(x_ref, w_ref, b_ref, o_ref):
    o_ref[...] = (
        jnp.dot(x_ref[...], w_ref[...], preferred_element_type=jnp.float32)
        + b_ref[...]
    )


def _matmul(x, w_t, b, blk=512):
    n, k = x.shape
    m = w_t.shape[1]
    return pl.pallas_call(
        _matmul_body,
        grid=(n // blk,),
        in_specs=[
            pl.BlockSpec((blk, k), lambda i: (i, 0)),
            pl.BlockSpec((k, m), lambda i: (0, 0)),
            pl.BlockSpec((1, m), lambda i: (0, 0)),
        ],
        out_specs=pl.BlockSpec((blk, m), lambda i: (i, 0)),
        out_shape=jax.ShapeDtypeStruct((n, m), jnp.float32),
    )(x, w_t, b)


_NBLK_ACT = N_ACT // 512
_NBLK_TOT = TOT // 512


def _fused_body(act_ref, pas_ref, ids_ref, se_ref, vwt_ref, vb_ref,
                awt_ref, ab_ref, pa_ref, ps_ref, ah_ref,
                vals_ref, offs_ref, fw_ref):
    pid = pl.program_id(0)

    def pack_vals(x):
        vp = jnp.dot(x, vwt_ref[...], preferred_element_type=jnp.float32) + vb_ref[...]
        be = lax.bitcast_convert_type(vp[:, :128].astype(jnp.bfloat16), jnp.int16)
        bo = lax.bitcast_convert_type(vp[:, 128:].astype(jnp.bfloat16), jnp.int16)
        return (bo.astype(jnp.int32) << 16) | (be.astype(jnp.int32) & 0xFFFF)

    @pl.when(pid < _NBLK_ACT)
    def _():
        x = act_ref[...]
        vals_ref[...] = pack_vals(x)
        ids = ids_ref[...]
        r = x.shape[0]
        b = ids[:, 0:1]
        g = ids[:, 1:2]
        gx = ids[:, 2:3]
        gy = ids[:, 3:4]

        gb = jnp.broadcast_to(g, (r, FEAT))
        se = se_ref[...]
        af = x
        for mm in range(MAPS):
            af = af + jnp.where(
                gb == mm, jnp.broadcast_to(se[mm : mm + 1, :], (r, FEAT)), 0.0
            )

        l = jnp.dot(af, awt_ref[...], preferred_element_type=jnp.float32) + ab_ref[...]
        e = jnp.exp(l - jnp.max(l, axis=1, keepdims=True))
        ssum = (
            (e + pltpu.roll(e, 8, 1))
            + (pltpu.roll(e, 16, 1) + pltpu.roll(e, 24, 1))
        )
        aw = e / ssum

        lane = lax.broadcasted_iota(jnp.int32, (r, 128), 1)
        msl = lane % 16
        mj = msl // 4
        sj = msl % 4
        dx = sj % 2
        dy = sj // 2

        def selm(idx, vals):
            out = jnp.full((r, 128), vals[MAPS - 1], jnp.int32)
            for mm in range(MAPS - 1):
                out = jnp.where(idx == mm, vals[mm], out)
            return out

        smi = selm(mj, _S)
        base = selm(mj, _BASES)
        hw = smi * smi
        g128 = jnp.broadcast_to(g, (r, 128))
        sg = selm(g128, _S)
        sgf = sg.astype(jnp.float32)
        smf = smi.astype(jnp.float32)
        gxf = jnp.broadcast_to(gx, (r, 128)).astype(jnp.float32)
        gyf = jnp.broadcast_to(gy, (r, 128)).astype(jnp.float32)
        sx = (gxf + 0.5) / sgf * smf - 0.5
        sy = (gyf + 0.5) / sgf * smf - 0.5
        x0 = jnp.floor(sx).astype(jnp.int32)
        y0 = jnp.floor(sy).astype(jnp.int32)
        xi = x0 + dx
        yi = y0 + dy
        wx = 1.0 - jnp.abs(sx - xi.astype(jnp.float32))
        wy = 1.0 - jnp.abs(sy - yi.astype(jnp.float32))
        sw = wx * wy
        xc = jnp.clip(xi, 0, smi - 1)
        yc = jnp.clip(yi, 0, smi - 1)
        offs = base + jnp.broadcast_to(b, (r, 128)) * hw + yc * smi + xc
        offs_ref[...] = offs[:, :16]

        aw_p = jnp.dot(aw, pa_ref[...], preferred_element_type=jnp.float32,
                       precision=lax.Precision.HIGHEST)
        sw_p = jnp.dot(sw, ps_ref[...], preferred_element_type=jnp.float32,
                       precision=lax.Precision.HIGHEST)
        fw_ref[...] = aw_p * sw_p

    @pl.when(pid >= _NBLK_ACT)
    def _():
        vals_ref[...] = pack_vals(pas_ref[...])


def _fused_tc(act, pas, ids, se, vwt, vb, awt, ab):
    return pl.pallas_call(
        _fused_body,
        grid=(_NBLK_TOT,),
        in_specs=[
            pl.BlockSpec((512, FEAT), lambda i: (jnp.minimum(i, _NBLK_ACT - 1), 0)),
            pl.BlockSpec((512, FEAT), lambda i: (jnp.maximum(i - _NBLK_ACT, 0), 0)),
            pl.BlockSpec((512, 4), lambda i: (jnp.minimum(i, _NBLK_ACT - 1), 0)),
            pl.BlockSpec((MAPS, FEAT), lambda i: (0, 0)),
            pl.BlockSpec((FEAT, FEAT), lambda i: (0, 0)),
            pl.BlockSpec((1, FEAT), lambda i: (0, 0)),
            pl.BlockSpec((FEAT, 128), lambda i: (0, 0)),
            pl.BlockSpec((1, 128), lambda i: (0, 0)),
            pl.BlockSpec((128, 128), lambda i: (0, 0)),
            pl.BlockSpec((128, 128), lambda i: (0, 0)),
            pl.BlockSpec((128, 128), lambda i: (0, 0)),
        ],
        out_specs=[
            pl.BlockSpec((512, FEAT // 2), lambda i: (i, 0)),
            pl.BlockSpec((512, 16), lambda i: (jnp.minimum(i, _NBLK_ACT - 1), 0)),
            pl.BlockSpec((512, 128), lambda i: (jnp.minimum(i, _NBLK_ACT - 1), 0)),
        ],
        out_shape=[
            jax.ShapeDtypeStruct((TOT, FEAT // 2), jnp.int32),
            jax.ShapeDtypeStruct((N_ACT, 16), jnp.int32),
            jax.ShapeDtypeStruct((N_ACT, 128), jnp.float32),
        ],
    )(act, pas, ids, se, vwt, vb, awt, ab, _P_A, _P_S, _A_H)


def _sc_compute_chunk(rows_v, fw_v, out_v, out_hbm, fb, sem_o):
    for f in range(_C):
        def ms_body(msp, acc):
            accl = list(acc)
            fwvec = fw_v[f, pl.ds(msp * 16, 16)]
            for ms_off in range(2):
                rr = f * 16 + msp * 2 + ms_off
                for h in range(HEADS):
                    w = fwvec[ms_off * 8 + h]
                    xi = rows_v[rr, pl.ds(h * 16, 16)]
                    xe = plsc.bitcast(xi << 16, jnp.float32)
                    xo = plsc.bitcast(xi, jnp.float32)
                    accl[2 * h] = accl[2 * h] + w * xe
                    accl[2 * h + 1] = accl[2 * h + 1] + w * xo
            return tuple(accl)

        acc = lax.fori_loop(
            0, 8, ms_body,
            tuple(jnp.zeros((16,), jnp.float32) for _ in range(16)),
        )
        for v in range(16):
            out_v[f, pl.ds(v * 16, 16)] = acc[v]
    pltpu.async_copy(out_v, out_hbm.at[pl.ds(fb, _C)], sem_o)


def _sc_body(vals_hbm, fw_hbm, offs_hbm, idmap_hbm, out_hbm,
             idmap_v, offs_v, fid_v, fw_va, fw_vb, rows_va, rows_vb,
             out_va, out_vb,
             sem_ra, sem_rb, sem_fa, sem_fb, sem_oa, sem_ob):
    wid = lax.axis_index("s") * _NC + lax.axis_index("c")
    base = wid * _RPT
    pltpu.sync_copy(idmap_hbm, idmap_v)
    pltpu.sync_copy(offs_hbm.at[pl.ds(base * 16, _RPT * 16)], offs_v)

    def fid_loop(f, c):
        fid = plsc.load_gather(idmap_v, [offs_v[pl.ds(f * 16, 16)]])
        fid_v[pl.ds(f * 16, 16)] = fid
        return c

    lax.fori_loop(0, _RPT, fid_loop, 0)

    def start(k, rows_v, fw_v, sem_r, sem_f):
        pltpu.async_copy(
            vals_hbm.at[fid_v.at[pl.ds(k * (_C * 16), _C * 16)]], rows_v, sem_r
        )
        pltpu.async_copy(fw_hbm.at[pl.ds(base + k * _C, _C)], fw_v, sem_f)

    def wait(k, rows_v, fw_v, sem_r, sem_f):
        pltpu.make_async_copy(
            vals_hbm.at[fid_v.at[pl.ds(k * (_C * 16), _C * 16)]], rows_v, sem_r
        ).wait()
        pltpu.make_async_copy(
            fw_hbm.at[pl.ds(base + k * _C, _C)], fw_v, sem_f
        ).wait()

    def wait_out(k, out_v, sem_o):
        pltpu.make_async_copy(
            out_v, out_hbm.at[pl.ds(base + k * _C, _C)], sem_o
        ).wait()

    start(0, rows_va, fw_va, sem_ra, sem_fa)

    def pair(g, carry):
        ka = 2 * g
        kb = 2 * g + 1
        start(kb, rows_vb, fw_vb, sem_rb, sem_fb)
        wait(ka, rows_va, fw_va, sem_ra, sem_fa)

        @pl.when(g > 0)
        def _():
            wait_out(ka - 2, out_va, sem_oa)

        _sc_compute_chunk(rows_va, fw_va, out_va, out_hbm, base + ka * _C, sem_oa)

        @pl.when(g < _NCH // 2 - 1)
        def _():
            start(ka + 2, rows_va, fw_va, sem_ra, sem_fa)

        wait(kb, rows_vb, fw_vb, sem_rb, sem_fb)

        @pl.when(g > 0)
        def _():
            wait_out(kb - 2, out_vb, sem_ob)

        _sc_compute_chunk(rows_vb, fw_vb, out_vb, out_hbm, base + kb * _C, sem_ob)
        return carry

    lax.fori_loop(0, _NCH // 2, pair, 0)
    wait_out(_NCH - 2, out_va, sem_oa)
    wait_out(_NCH - 1, out_vb, sem_ob)


def _sc_gather_combine(all_vals, fw, offs, flat_idmap):
    mesh = plsc.VectorSubcoreMesh(
        core_axis_name="c", subcore_axis_name="s", num_cores=_NC, num_subcores=_NS
    )
    run = functools.partial(
        pl.kernel,
        out_type=jax.ShapeDtypeStruct((N_ACT, FEAT), jnp.float32),
        mesh=mesh,
        compiler_params=pltpu.CompilerParams(needs_layout_passes=False),
        scratch_types=[
            pltpu.VMEM((_IDTOT,), jnp.int32),
            pltpu.VMEM((_RPT * 16,), jnp.int32),
            pltpu.VMEM((_RPT * 16,), jnp.int32),
            pltpu.VMEM((_C, 128), jnp.float32),
            pltpu.VMEM((_C, 128), jnp.float32),
            pltpu.VMEM((_C * 16, FEAT // 2), jnp.int32),
            pltpu.VMEM((_C * 16, FEAT // 2), jnp.int32),
            pltpu.VMEM((_C, FEAT), jnp.float32),
            pltpu.VMEM((_C, FEAT), jnp.float32),
            pltpu.SemaphoreType.DMA,
            pltpu.SemaphoreType.DMA,
            pltpu.SemaphoreType.DMA,
            pltpu.SemaphoreType.DMA,
            pltpu.SemaphoreType.DMA,
            pltpu.SemaphoreType.DMA,
        ],
    )(_sc_body)
    return run(all_vals, fw, offs, flat_idmap)


def kernel(in_act_feats, act_batch_ids, act_map_ids, act_xy_ids, map_shapes,
           pas_feats, id_map0, id_map1, id_map2, id_map3, scale_embed,
           attn_W, attn_b, val_W, val_b, out_W, out_b):
    del map_shapes
    ids = jnp.stack(
        [act_batch_ids, act_map_ids, act_xy_ids[:, 0], act_xy_ids[:, 1]], axis=1
    )
    attn_wr_t = attn_W.reshape(HEADS, MAPS, FEAT).transpose(1, 0, 2).reshape(
        HEADS * MAPS, FEAT).T
    awt128 = jnp.concatenate([attn_wr_t] * 4, axis=1)
    attn_br = attn_b.reshape(HEADS, MAPS).T.reshape(1, HEADS * MAPS)
    ab128 = jnp.concatenate([attn_br] * 4, axis=1)
    flat_idmap = jnp.concatenate(
        [m.reshape(-1) for m in (id_map0, id_map1, id_map2, id_map3)]
    )
    vwt_perm = val_W.T.reshape(FEAT, 128, 2).transpose(0, 2, 1).reshape(FEAT, FEAT)
    vb_perm = val_b.reshape(128, 2).transpose(1, 0).reshape(1, FEAT)
    vals_i32, offs, fw = _fused_tc(
        in_act_feats, pas_feats, ids, scale_embed, vwt_perm, vb_perm, awt128, ab128,
    )
    val_feats = _sc_gather_combine(vals_i32, fw, offs.reshape(-1), flat_idmap)
    out_wt_perm = out_W.T.reshape(8, 16, 2, FEAT).transpose(0, 2, 1, 3).reshape(
        FEAT, FEAT)
    return _matmul(val_feats, out_wt_perm, out_b.reshape(1, FEAT))
